# Initial kernel scaffold; baseline (speedup 1.0000x reference)
import jax
import jax.numpy as jnp
from jax.experimental import pallas as pl


def kernel(x, gates_w, shared_fc1_w, shared_fc1_b, shared_fc2_w, shared_fc2_b, task_fc1_w, task_fc1_b, task_fc2_w, task_fc2_b):
    raise NotImplementedError("write your pallas kernel here")


# speedup vs baseline: 1.4727x; 1.4727x over previous
import functools

import jax
import jax.numpy as jnp
from jax.experimental import pallas as pl
from jax.experimental.pallas import tpu as pltpu

N = 2048
D = 768
H = 3072
E = 5
HBLK = 512
NHB = H // HBLK
NEG = -3.0e38


def _routing(x_ref, g_ref, wsc_ref, aux0_ref, aux1_ref):
    glog = jnp.dot(x_ref[...], g_ref[...], preferred_element_type=jnp.float32)
    iota = jax.lax.broadcasted_iota(jnp.int32, (N, 5), 1)
    for t in range(2):
        l = glog[:, 5 * t:5 * t + 5]
        m1 = jnp.max(l, axis=-1, keepdims=True)
        idx1 = jnp.min(jnp.where(l == m1, iota, 5), axis=-1, keepdims=True)
        sel1 = iota == idx1
        l2 = jnp.where(sel1, NEG, l)
        m2 = jnp.max(l2, axis=-1, keepdims=True)
        idx2 = jnp.min(jnp.where(l2 == m2, iota, 5), axis=-1, keepdims=True)
        sel = sel1 | (iota == idx2)
        denom = 1.0 + jnp.exp(m2 - m1)
        w = jnp.where(sel, jnp.exp(l - m1), 0.0) / denom
        p = jnp.exp(l - m1)
        p = p / jnp.sum(p, axis=-1, keepdims=True)
        f = jnp.sum(sel.astype(jnp.float32), axis=0, keepdims=True) / N
        pbar = jnp.sum(p, axis=0, keepdims=True) / N
        aux = jnp.sum(f * pbar, keepdims=True).reshape(1, 1)
        if t == 0:
            aux0_ref[...] = aux
            wsc_ref[:, 0:3] = w[:, 0:3]
        else:
            aux1_ref[...] = aux
            wsc_ref[:, 5:6] = w[:, 0:1]
            wsc_ref[:, 3:5] = w[:, 1:3]


def _moe_kernel(x_ref, xb_ref, g_ref, w1_ref, b1_ref, w2_ref, b2_ref,
                out0_ref, out1_ref, aux0_ref, aux1_ref, wsc_ref):
    e = pl.program_id(0)
    hb = pl.program_id(1)

    @pl.when((e == 0) & (hb == 0))
    def _init():
        _routing(x_ref, g_ref, wsc_ref, aux0_ref, aux1_ref)
        out0_ref[...] = jnp.zeros_like(out0_ref)
        out1_ref[...] = jnp.zeros_like(out1_ref)

    h = jnp.dot(xb_ref[...], w1_ref[0], preferred_element_type=jnp.float32)
    h = h + b1_ref[0]
    h = 0.5 * h * (1.0 + jax.lax.erf(h * 0.7071067811865476))
    partial = jnp.dot(h.astype(jnp.bfloat16), w2_ref[0],
                      preferred_element_type=jnp.float32)
    partial = partial + b2_ref[0] * jnp.where(hb == 0, 1.0, 0.0)

    @pl.when(e == 0)
    def _shared():
        out0_ref[...] += partial * wsc_ref[:, 0:1]
        out1_ref[...] += partial * wsc_ref[:, 5:6]

    @pl.when(e == 1)
    def _t0e0():
        out0_ref[...] += partial * wsc_ref[:, 1:2]

    @pl.when(e == 2)
    def _t0e1():
        out0_ref[...] += partial * wsc_ref[:, 2:3]

    @pl.when(e == 3)
    def _t1e0():
        out1_ref[...] += partial * wsc_ref[:, 3:4]

    @pl.when(e == 4)
    def _t1e1():
        out1_ref[...] += partial * wsc_ref[:, 4:5]


@functools.partial(jax.jit, static_argnames=("interpret",))
def _run(x, gates_w, shared_fc1_w, shared_fc1_b, shared_fc2_w, shared_fc2_b,
         task_fc1_w, task_fc1_b, task_fc2_w, task_fc2_b, interpret=False):
    x2 = x.reshape(N, D)
    w1 = jnp.concatenate([shared_fc1_w, task_fc1_w.reshape(4, H, D)], axis=0)
    w2 = jnp.concatenate([shared_fc2_w, task_fc2_w.reshape(4, D, H)], axis=0)
    b1 = jnp.concatenate([shared_fc1_b, task_fc1_b.reshape(4, H)], axis=0)
    b2 = jnp.concatenate([shared_fc2_b, task_fc2_b.reshape(4, D)], axis=0)
    w1t = jnp.transpose(w1, (0, 2, 1)).astype(jnp.bfloat16)
    w2t = jnp.transpose(w2, (0, 2, 1)).astype(jnp.bfloat16)
    xb = x2.astype(jnp.bfloat16)
    g = jnp.transpose(gates_w, (2, 0, 1)).reshape(D, 10)

    out0, out1, aux0, aux1 = pl.pallas_call(
        _moe_kernel,
        grid=(E, NHB),
        in_specs=[
            pl.BlockSpec((N, D), lambda e, h: (0, 0)),
            pl.BlockSpec((N, D), lambda e, h: (0, 0)),
            pl.BlockSpec((D, 10), lambda e, h: (0, 0)),
            pl.BlockSpec((1, D, HBLK), lambda e, h: (e, 0, h)),
            pl.BlockSpec((1, 1, HBLK), lambda e, h: (e, 0, h)),
            pl.BlockSpec((1, HBLK, D), lambda e, h: (e, h, 0)),
            pl.BlockSpec((1, 1, D), lambda e, h: (e, 0, 0)),
        ],
        out_specs=[
            pl.BlockSpec((N, D), lambda e, h: (0, 0)),
            pl.BlockSpec((N, D), lambda e, h: (0, 0)),
            pl.BlockSpec((1, 1), lambda e, h: (0, 0)),
            pl.BlockSpec((1, 1), lambda e, h: (0, 0)),
        ],
        out_shape=[
            jax.ShapeDtypeStruct((N, D), jnp.float32),
            jax.ShapeDtypeStruct((N, D), jnp.float32),
            jax.ShapeDtypeStruct((1, 1), jnp.float32),
            jax.ShapeDtypeStruct((1, 1), jnp.float32),
        ],
        scratch_shapes=[pltpu.VMEM((N, 8), jnp.float32)],
        compiler_params=pltpu.CompilerParams(
            dimension_semantics=("arbitrary", "arbitrary"),
        ),
        interpret=interpret,
    )(x2, xb, g, w1t, b1.reshape(E, 1, H), w2t, b2.reshape(E, 1, D))

    out0 = out0.reshape(1, N, D)
    out1 = out1.reshape(1, N, D)
    return out0, aux0[0, 0], out1, aux1[0, 0]


def kernel(x, gates_w, shared_fc1_w, shared_fc1_b, shared_fc2_w, shared_fc2_b,
           task_fc1_w, task_fc1_b, task_fc2_w, task_fc2_b):
    return _run(x, gates_w, shared_fc1_w, shared_fc1_b, shared_fc2_w,
                shared_fc2_b, task_fc1_w, task_fc1_b, task_fc2_w, task_fc2_b)


# speedup vs baseline: 1.6090x; 1.0926x over previous
import functools

import jax
import jax.numpy as jnp
from jax.experimental import pallas as pl
from jax.experimental.pallas import tpu as pltpu

N = 2048
D = 768
H = 3072
E = 5
NB = 512
NNB = N // NB
NEG = -3.0e38


def _router_kernel(x_ref, g_ref, wsc_ref, aux0_ref, aux1_ref):
    glog = jnp.dot(x_ref[...], g_ref[...], preferred_element_type=jnp.float32)
    iota = jax.lax.broadcasted_iota(jnp.int32, (N, 5), 1)
    for t in range(2):
        l = glog[:, 5 * t:5 * t + 5]
        m1 = jnp.max(l, axis=-1, keepdims=True)
        idx1 = jnp.min(jnp.where(l == m1, iota, 5), axis=-1, keepdims=True)
        sel1 = iota == idx1
        l2 = jnp.where(sel1, NEG, l)
        m2 = jnp.max(l2, axis=-1, keepdims=True)
        idx2 = jnp.min(jnp.where(l2 == m2, iota, 5), axis=-1, keepdims=True)
        sel = sel1 | (iota == idx2)
        denom = 1.0 + jnp.exp(m2 - m1)
        w = jnp.where(sel, jnp.exp(l - m1), 0.0) / denom
        p = jnp.exp(l - m1)
        p = p / jnp.sum(p, axis=-1, keepdims=True)
        f = jnp.sum(sel.astype(jnp.float32), axis=0, keepdims=True) / N
        pbar = jnp.sum(p, axis=0, keepdims=True) / N
        aux = jnp.sum(f * pbar, keepdims=True).reshape(1, 1)
        if t == 0:
            aux0_ref[...] = aux
            wsc_ref[:, 0:3] = w[:, 0:3]
        else:
            aux1_ref[...] = aux
            wsc_ref[:, 5:6] = w[:, 0:1]
            wsc_ref[:, 3:5] = w[:, 1:3]
            wsc_ref[:, 6:8] = jnp.zeros((N, 2), jnp.float32)


def _expert_kernel(xb_ref, wsc_ref, w1_ref, b1_ref, w2_ref, b2_ref,
                   out0_ref, out1_ref):
    e = pl.program_id(0)
    nb = pl.program_id(1)
    rows = pl.ds(nb * NB, NB)
    h = jnp.dot(xb_ref[rows, :], w1_ref[0],
                preferred_element_type=jnp.float32)
    h = h + b1_ref[0]
    h = 0.5 * h * (1.0 + jax.lax.erf(h * 0.7071067811865476))
    partial = jnp.dot(h.astype(jnp.bfloat16), w2_ref[0],
                      preferred_element_type=jnp.float32) + b2_ref[0]

    @pl.when(e == 0)
    def _shared():
        out0_ref[rows, :] = partial * wsc_ref[rows, 0:1]
        out1_ref[rows, :] = partial * wsc_ref[rows, 5:6]

    @pl.when(e == 1)
    def _t0e0():
        out0_ref[rows, :] += partial * wsc_ref[rows, 1:2]

    @pl.when(e == 2)
    def _t0e1():
        out0_ref[rows, :] += partial * wsc_ref[rows, 2:3]

    @pl.when(e == 3)
    def _t1e0():
        out1_ref[rows, :] += partial * wsc_ref[rows, 3:4]

    @pl.when(e == 4)
    def _t1e1():
        out1_ref[rows, :] += partial * wsc_ref[rows, 4:5]


@functools.partial(jax.jit, static_argnames=("interpret",))
def _run(x, gates_w, shared_fc1_w, shared_fc1_b, shared_fc2_w, shared_fc2_b,
         task_fc1_w, task_fc1_b, task_fc2_w, task_fc2_b, interpret=False):
    x2 = x.reshape(N, D)
    w1 = jnp.concatenate([shared_fc1_w, task_fc1_w.reshape(4, H, D)], axis=0)
    w2 = jnp.concatenate([shared_fc2_w, task_fc2_w.reshape(4, D, H)], axis=0)
    b1 = jnp.concatenate([shared_fc1_b, task_fc1_b.reshape(4, H)], axis=0)
    b2 = jnp.concatenate([shared_fc2_b, task_fc2_b.reshape(4, D)], axis=0)
    w1t = jnp.transpose(w1, (0, 2, 1)).astype(jnp.bfloat16)
    w2t = jnp.transpose(w2, (0, 2, 1)).astype(jnp.bfloat16)
    xb = x2.astype(jnp.bfloat16)
    g = jnp.transpose(gates_w, (2, 0, 1)).reshape(D, 10)

    wsc, aux0, aux1 = pl.pallas_call(
        _router_kernel,
        grid=(1,),
        in_specs=[
            pl.BlockSpec((N, D), lambda i: (0, 0)),
            pl.BlockSpec((D, 10), lambda i: (0, 0)),
        ],
        out_specs=[
            pl.BlockSpec((N, 8), lambda i: (0, 0)),
            pl.BlockSpec((1, 1), lambda i: (0, 0)),
            pl.BlockSpec((1, 1), lambda i: (0, 0)),
        ],
        out_shape=[
            jax.ShapeDtypeStruct((N, 8), jnp.float32),
            jax.ShapeDtypeStruct((1, 1), jnp.float32),
            jax.ShapeDtypeStruct((1, 1), jnp.float32),
        ],
        interpret=interpret,
    )(x2, g)

    out0, out1 = pl.pallas_call(
        _expert_kernel,
        grid=(E, NNB),
        in_specs=[
            pl.BlockSpec((N, D), lambda e, n: (0, 0)),
            pl.BlockSpec((N, 8), lambda e, n: (0, 0)),
            pl.BlockSpec((1, D, H), lambda e, n: (e, 0, 0)),
            pl.BlockSpec((1, 1, H), lambda e, n: (e, 0, 0)),
            pl.BlockSpec((1, H, D), lambda e, n: (e, 0, 0)),
            pl.BlockSpec((1, 1, D), lambda e, n: (e, 0, 0)),
        ],
        out_specs=[
            pl.BlockSpec((N, D), lambda e, n: (0, 0)),
            pl.BlockSpec((N, D), lambda e, n: (0, 0)),
        ],
        out_shape=[
            jax.ShapeDtypeStruct((N, D), jnp.float32),
            jax.ShapeDtypeStruct((N, D), jnp.float32),
        ],
        compiler_params=pltpu.CompilerParams(
            dimension_semantics=("arbitrary", "arbitrary"),
            vmem_limit_bytes=110 * 1024 * 1024,
        ),
        interpret=interpret,
    )(xb, wsc, w1t, b1.reshape(E, 1, H), w2t, b2.reshape(E, 1, D))

    out0 = out0.reshape(1, N, D)
    out1 = out1.reshape(1, N, D)
    return out0, aux0[0, 0], out1, aux1[0, 0]


def kernel(x, gates_w, shared_fc1_w, shared_fc1_b, shared_fc2_w, shared_fc2_b,
           task_fc1_w, task_fc1_b, task_fc2_w, task_fc2_b):
    return _run(x, gates_w, shared_fc1_w, shared_fc1_b, shared_fc2_w,
                shared_fc2_b, task_fc1_w, task_fc1_b, task_fc2_w, task_fc2_b)


# speedup vs baseline: 2.0399x; 1.2678x over previous
import functools

import jax
import jax.numpy as jnp
from jax.experimental import pallas as pl
from jax.experimental.pallas import tpu as pltpu

N = 2048
D = 768
H = 3072
E = 5
NB = 512
NNB = N // NB
NEG = -3.0e38


def _nt_dot(a, b):
    return jax.lax.dot_general(a, b, (((1,), (1,)), ((), ())),
                               preferred_element_type=jnp.float32)


def _router_kernel(x_ref, g_ref, wsc_ref, aux0_ref, aux1_ref):
    glog = _nt_dot(x_ref[...], g_ref[...])
    iota = jax.lax.broadcasted_iota(jnp.int32, (N, 5), 1)
    for t in range(2):
        l = glog[:, 5 * t:5 * t + 5]
        m1 = jnp.max(l, axis=-1, keepdims=True)
        idx1 = jnp.min(jnp.where(l == m1, iota, 5), axis=-1, keepdims=True)
        sel1 = iota == idx1
        l2 = jnp.where(sel1, NEG, l)
        m2 = jnp.max(l2, axis=-1, keepdims=True)
        idx2 = jnp.min(jnp.where(l2 == m2, iota, 5), axis=-1, keepdims=True)
        sel = sel1 | (iota == idx2)
        denom = 1.0 + jnp.exp(m2 - m1)
        w = jnp.where(sel, jnp.exp(l - m1), 0.0) / denom
        p = jnp.exp(l - m1)
        p = p / jnp.sum(p, axis=-1, keepdims=True)
        f = jnp.sum(sel.astype(jnp.float32), axis=0, keepdims=True) / N
        pbar = jnp.sum(p, axis=0, keepdims=True) / N
        aux = jnp.sum(f * pbar, keepdims=True).reshape(1, 1)
        if t == 0:
            aux0_ref[...] = aux
            wsc_ref[:, 0:3] = w[:, 0:3]
        else:
            aux1_ref[...] = aux
            wsc_ref[:, 5:6] = w[:, 0:1]
            wsc_ref[:, 3:5] = w[:, 1:3]
            wsc_ref[:, 6:8] = jnp.zeros((N, 2), jnp.float32)


def _expert_kernel(xb_ref, wsc_ref, w1_ref, b1_ref, w2_ref, b2_ref,
                   out0_ref, out1_ref):
    e = pl.program_id(0)
    nb = pl.program_id(1)
    rows = pl.ds(nb * NB, NB)
    h = _nt_dot(xb_ref[rows, :], w1_ref[0])
    h = h + b1_ref[0]
    h = 0.5 * h * (1.0 + jax.lax.erf(h * 0.7071067811865476))
    partial = _nt_dot(h.astype(jnp.bfloat16), w2_ref[0]) + b2_ref[0]

    @pl.when(e == 0)
    def _shared():
        out0_ref[rows, :] = partial * wsc_ref[rows, 0:1]
        out1_ref[rows, :] = partial * wsc_ref[rows, 5:6]

    @pl.when(e == 1)
    def _t0e0():
        out0_ref[rows, :] += partial * wsc_ref[rows, 1:2]

    @pl.when(e == 2)
    def _t0e1():
        out0_ref[rows, :] += partial * wsc_ref[rows, 2:3]

    @pl.when(e == 3)
    def _t1e0():
        out1_ref[rows, :] += partial * wsc_ref[rows, 3:4]

    @pl.when(e == 4)
    def _t1e1():
        out1_ref[rows, :] += partial * wsc_ref[rows, 4:5]


@functools.partial(jax.jit, static_argnames=("interpret",))
def _run(x, gates_w, shared_fc1_w, shared_fc1_b, shared_fc2_w, shared_fc2_b,
         task_fc1_w, task_fc1_b, task_fc2_w, task_fc2_b, interpret=False):
    x2 = x.reshape(N, D)
    w1 = jnp.concatenate([shared_fc1_w, task_fc1_w.reshape(4, H, D)], axis=0)
    w2 = jnp.concatenate([shared_fc2_w, task_fc2_w.reshape(4, D, H)], axis=0)
    b1 = jnp.concatenate([shared_fc1_b, task_fc1_b.reshape(4, H)], axis=0)
    b2 = jnp.concatenate([shared_fc2_b, task_fc2_b.reshape(4, D)], axis=0)
    w1t = w1.astype(jnp.bfloat16)
    w2t = w2.astype(jnp.bfloat16)
    xb = x2.astype(jnp.bfloat16)
    g = gates_w.reshape(10, D)

    wsc, aux0, aux1 = pl.pallas_call(
        _router_kernel,
        grid=(1,),
        in_specs=[
            pl.BlockSpec((N, D), lambda i: (0, 0)),
            pl.BlockSpec((10, D), lambda i: (0, 0)),
        ],
        out_specs=[
            pl.BlockSpec((N, 8), lambda i: (0, 0)),
            pl.BlockSpec((1, 1), lambda i: (0, 0)),
            pl.BlockSpec((1, 1), lambda i: (0, 0)),
        ],
        out_shape=[
            jax.ShapeDtypeStruct((N, 8), jnp.float32),
            jax.ShapeDtypeStruct((1, 1), jnp.float32),
            jax.ShapeDtypeStruct((1, 1), jnp.float32),
        ],
        interpret=interpret,
    )(x2, g)

    out0, out1 = pl.pallas_call(
        _expert_kernel,
        grid=(E, NNB),
        in_specs=[
            pl.BlockSpec((N, D), lambda e, n: (0, 0)),
            pl.BlockSpec((N, 8), lambda e, n: (0, 0)),
            pl.BlockSpec((1, H, D), lambda e, n: (e, 0, 0)),
            pl.BlockSpec((1, 1, H), lambda e, n: (e, 0, 0)),
            pl.BlockSpec((1, D, H), lambda e, n: (e, 0, 0)),
            pl.BlockSpec((1, 1, D), lambda e, n: (e, 0, 0)),
        ],
        out_specs=[
            pl.BlockSpec((N, D), lambda e, n: (0, 0)),
            pl.BlockSpec((N, D), lambda e, n: (0, 0)),
        ],
        out_shape=[
            jax.ShapeDtypeStruct((N, D), jnp.float32),
            jax.ShapeDtypeStruct((N, D), jnp.float32),
        ],
        compiler_params=pltpu.CompilerParams(
            dimension_semantics=("arbitrary", "arbitrary"),
            vmem_limit_bytes=110 * 1024 * 1024,
        ),
        interpret=interpret,
    )(xb, wsc, w1t, b1.reshape(E, 1, H), w2t, b2.reshape(E, 1, D))

    out0 = out0.reshape(1, N, D)
    out1 = out1.reshape(1, N, D)
    return out0, aux0[0, 0], out1, aux1[0, 0]


def kernel(x, gates_w, shared_fc1_w, shared_fc1_b, shared_fc2_w, shared_fc2_b,
           task_fc1_w, task_fc1_b, task_fc2_w, task_fc2_b):
    return _run(x, gates_w, shared_fc1_w, shared_fc1_b, shared_fc2_w,
                shared_fc2_b, task_fc1_w, task_fc1_b, task_fc2_w, task_fc2_b)


# speedup vs baseline: 2.6872x; 1.3173x over previous
import functools

import jax
import jax.numpy as jnp
from jax.experimental import pallas as pl
from jax.experimental.pallas import tpu as pltpu

N = 2048
D = 768
H = 3072
NB = 512
NNB = N // NB
HSPLIT = 2
HS = H // HSPLIT
NEG = -3.0e38


def _nt_dot(a, b):
    return jax.lax.dot_general(a, b, (((1,), (1,)), ((), ())),
                               preferred_element_type=jnp.float32)


def _ffn(xrow, w1f, b1_ref, w2f, b2_ref):
    partial = b2_ref[0]
    for s in range(HSPLIT):
        lo, hi = s * HS, (s + 1) * HS
        h = _nt_dot(xrow, w1f[lo:hi, :].astype(jnp.bfloat16))
        h = h + b1_ref[0, :, lo:hi]
        h = 0.5 * h * (1.0 + jax.lax.erf(h * 0.7071067811865476))
        partial = partial + _nt_dot(h.astype(jnp.bfloat16),
                                    w2f[:, lo:hi].astype(jnp.bfloat16))
    return partial


def _router_kernel(x_ref, g_ref, wsc_ref, aux0_ref, aux1_ref, xb_ref):
    xb_ref[...] = x_ref[...].astype(jnp.bfloat16)
    glog = _nt_dot(x_ref[...], g_ref[...])
    iota = jax.lax.broadcasted_iota(jnp.int32, (N, 5), 1)
    for t in range(2):
        l = glog[:, 5 * t:5 * t + 5]
        m1 = jnp.max(l, axis=-1, keepdims=True)
        idx1 = jnp.min(jnp.where(l == m1, iota, 5), axis=-1, keepdims=True)
        sel1 = iota == idx1
        l2 = jnp.where(sel1, NEG, l)
        m2 = jnp.max(l2, axis=-1, keepdims=True)
        idx2 = jnp.min(jnp.where(l2 == m2, iota, 5), axis=-1, keepdims=True)
        sel = sel1 | (iota == idx2)
        denom = 1.0 + jnp.exp(m2 - m1)
        w = jnp.where(sel, jnp.exp(l - m1), 0.0) / denom
        p = jnp.exp(l - m1)
        p = p / jnp.sum(p, axis=-1, keepdims=True)
        f = jnp.sum(sel.astype(jnp.float32), axis=0, keepdims=True) / N
        pbar = jnp.sum(p, axis=0, keepdims=True) / N
        aux = jnp.sum(f * pbar, keepdims=True).reshape(1, 1)
        if t == 0:
            aux0_ref[...] = aux
            wsc_ref[:, 0:3] = w[:, 0:3]
        else:
            aux1_ref[...] = aux
            wsc_ref[:, 5:6] = w[:, 0:1]
            wsc_ref[:, 3:5] = w[:, 1:3]
            wsc_ref[:, 6:8] = jnp.zeros((N, 2), jnp.float32)


def _task_kernel(xb_ref, wsc_ref, t1_ref, b1_ref, t2_ref, b2_ref,
                 tout0_ref, tout1_ref):
    e = pl.program_id(0)
    nb = pl.program_id(1)
    rows = pl.ds(nb * NB, NB)
    partial = _ffn(xb_ref[rows, :], t1_ref[0], b1_ref, t2_ref[0], b2_ref)

    @pl.when(e == 0)
    def _t0e0():
        tout0_ref[rows, :] = partial * wsc_ref[rows, 1:2]

    @pl.when(e == 1)
    def _t0e1():
        tout0_ref[rows, :] += partial * wsc_ref[rows, 2:3]

    @pl.when(e == 2)
    def _t1e0():
        tout1_ref[rows, :] = partial * wsc_ref[rows, 3:4]

    @pl.when(e == 3)
    def _t1e1():
        tout1_ref[rows, :] += partial * wsc_ref[rows, 4:5]


def _shared_kernel(xb_ref, wsc_ref, s1_ref, b1_ref, s2_ref, b2_ref,
                   tout0_ref, tout1_ref, out0_ref, out1_ref):
    nb = pl.program_id(0)
    rows = pl.ds(nb * NB, NB)
    partial = _ffn(xb_ref[rows, :], s1_ref[0], b1_ref, s2_ref[0], b2_ref)
    out0_ref[rows, :] = tout0_ref[rows, :] + partial * wsc_ref[rows, 0:1]
    out1_ref[rows, :] = tout1_ref[rows, :] + partial * wsc_ref[rows, 5:6]


@functools.partial(jax.jit, static_argnames=("interpret",))
def _run(x, gates_w, shared_fc1_w, shared_fc1_b, shared_fc2_w, shared_fc2_b,
         task_fc1_w, task_fc1_b, task_fc2_w, task_fc2_b, interpret=False):
    x2 = x.reshape(N, D)
    t1 = task_fc1_w.reshape(4, H, D)
    t2 = task_fc2_w.reshape(4, D, H)
    t1b = task_fc1_b.reshape(4, 1, H)
    t2b = task_fc2_b.reshape(4, 1, D)
    g = gates_w.reshape(10, D)

    wsc, aux0, aux1, xb = pl.pallas_call(
        _router_kernel,
        grid=(1,),
        in_specs=[
            pl.BlockSpec((N, D), lambda i: (0, 0)),
            pl.BlockSpec((10, D), lambda i: (0, 0)),
        ],
        out_specs=[
            pl.BlockSpec((N, 8), lambda i: (0, 0)),
            pl.BlockSpec((1, 1), lambda i: (0, 0)),
            pl.BlockSpec((1, 1), lambda i: (0, 0)),
            pl.BlockSpec((N, D), lambda i: (0, 0)),
        ],
        out_shape=[
            jax.ShapeDtypeStruct((N, 8), jnp.float32),
            jax.ShapeDtypeStruct((1, 1), jnp.float32),
            jax.ShapeDtypeStruct((1, 1), jnp.float32),
            jax.ShapeDtypeStruct((N, D), jnp.bfloat16),
        ],
        interpret=interpret,
    )(x2, g)

    tout0, tout1 = pl.pallas_call(
        _task_kernel,
        grid=(4, NNB),
        in_specs=[
            pl.BlockSpec((N, D), lambda e, n: (0, 0)),
            pl.BlockSpec((N, 8), lambda e, n: (0, 0)),
            pl.BlockSpec((1, H, D), lambda e, n: (e, 0, 0)),
            pl.BlockSpec((1, 1, H), lambda e, n: (e, 0, 0)),
            pl.BlockSpec((1, D, H), lambda e, n: (e, 0, 0)),
            pl.BlockSpec((1, 1, D), lambda e, n: (e, 0, 0)),
        ],
        out_specs=[
            pl.BlockSpec((N, D), lambda e, n: (0, 0)),
            pl.BlockSpec((N, D), lambda e, n: (0, 0)),
        ],
        out_shape=[
            jax.ShapeDtypeStruct((N, D), jnp.float32),
            jax.ShapeDtypeStruct((N, D), jnp.float32),
        ],
        compiler_params=pltpu.CompilerParams(
            dimension_semantics=("arbitrary", "arbitrary"),
            vmem_limit_bytes=63 * 1024 * 1024,
        ),
        interpret=interpret,
    )(xb, wsc, t1, t1b, t2, t2b)

    out0, out1 = pl.pallas_call(
        _shared_kernel,
        grid=(NNB,),
        in_specs=[
            pl.BlockSpec((N, D), lambda n: (0, 0)),
            pl.BlockSpec((N, 8), lambda n: (0, 0)),
            pl.BlockSpec((1, H, D), lambda n: (0, 0, 0)),
            pl.BlockSpec((1, 1, H), lambda n: (0, 0, 0)),
            pl.BlockSpec((1, D, H), lambda n: (0, 0, 0)),
            pl.BlockSpec((1, 1, D), lambda n: (0, 0, 0)),
            pl.BlockSpec((N, D), lambda n: (0, 0)),
            pl.BlockSpec((N, D), lambda n: (0, 0)),
        ],
        out_specs=[
            pl.BlockSpec((N, D), lambda n: (0, 0)),
            pl.BlockSpec((N, D), lambda n: (0, 0)),
        ],
        out_shape=[
            jax.ShapeDtypeStruct((N, D), jnp.float32),
            jax.ShapeDtypeStruct((N, D), jnp.float32),
        ],
        compiler_params=pltpu.CompilerParams(
            dimension_semantics=("arbitrary",),
            vmem_limit_bytes=63 * 1024 * 1024,
        ),
        interpret=interpret,
    )(xb, wsc, shared_fc1_w, shared_fc1_b.reshape(1, 1, H),
      shared_fc2_w, shared_fc2_b.reshape(1, 1, D), tout0, tout1)

    out0 = out0.reshape(1, N, D)
    out1 = out1.reshape(1, N, D)
    return out0, aux0[0, 0], out1, aux1[0, 0]


def kernel(x, gates_w, shared_fc1_w, shared_fc1_b, shared_fc2_w, shared_fc2_b,
           task_fc1_w, task_fc1_b, task_fc2_w, task_fc2_b):
    return _run(x, gates_w, shared_fc1_w, shared_fc1_b, shared_fc2_w,
                shared_fc2_b, task_fc1_w, task_fc1_b, task_fc2_w, task_fc2_b)
